# baseline (speedup 1.0000x reference)
import functools

import jax
import jax.numpy as jnp
from jax import lax
from jax.experimental import pallas as pl
from jax.experimental.pallas import tpu as pltpu
from jax.experimental.pallas import tpu_sc as plsc

_N = 10000
_E = 320000
_H = 64
_NPAD = 10016
_EPAD = 327680
_NTILES = 32
_CPC = 160
_BLK = 1000


def _mesh():
    return plsc.VectorSubcoreMesh(core_axis_name="c", subcore_axis_name="s")


_SC_PARAMS = pltpu.CompilerParams(use_tc_tiling_on_sc=False)


def _sc_relation(h2, dmeta, src3, dst3, rel):
    @functools.partial(
        pl.kernel,
        out_type=(
            jax.ShapeDtypeStruct((2, _N, 32), jnp.float32),
            jax.ShapeDtypeStruct((2, _N, 32), jnp.float32),
        ),
        mesh=_mesh(),
        scratch_types=[
            pltpu.VMEM((160, 128), jnp.int32),
            pltpu.VMEM((160, 128), jnp.int32),
            pltpu.VMEM((128, 32), jnp.float32),
            pltpu.VMEM((128, 32), jnp.float32),
            pltpu.VMEM((128, 32), jnp.float32),
            pltpu.VMEM((128, 32), jnp.float32),
            pltpu.VMEM((128, 32), jnp.float32),
            pltpu.VMEM((128, 32), jnp.float32),
            pltpu.VMEM((128, 32), jnp.float32),
            pltpu.VMEM((128, 32), jnp.float32),
            pltpu.VMEM_SHARED((_NPAD, 32), jnp.float32),
            pltpu.VMEM_SHARED((_N, 32), jnp.float32),
        ] + [pltpu.SemaphoreType.DMA] * 16,
        compiler_params=_SC_PARAMS,
    )
    def k(h_hbm, dm_hbm, src_hbm, dst_hbm, f1_hbm, m2_hbm, sidx, didx,
          r0, r1, r2, r3, r4, r5, r6, r7, acc, ycopy, *sems):
        cid = lax.axis_index("c")
        sid = lax.axis_index("s")
        rows = (r0, r1, r2, r3, r4, r5, r6, r7)
        gsem = sems[:8]
        ssem = sems[8:]
        zero = jnp.zeros((16,), jnp.float32)
        base = sid * 624

        def row_chunks(f):
            for t in range(4):
                f(base + t * 128, 128)
            f(base + 512, 112)

            @pl.when(sid == 15)
            def _():
                f(9984, 16)

        def fill_zero(buf):
            @pl.loop(0, 128)
            def _(r):
                for c2 in range(2):
                    buf[r, pl.ds(c2 * 16, 16)] = zero

        def zero_acc():
            fill_zero(r0)

            def zc(row0, sz):
                pltpu.sync_copy(r0.at[pl.ds(0, sz)], acc.at[pl.ds(row0, sz)])

            row_chunks(zc)

        zero_acc()

        def y0_chunk(row0, sz):
            pltpu.sync_copy(h_hbm.at[cid, pl.ds(row0, sz)], r1.at[pl.ds(0, sz)])
            pltpu.sync_copy(dm_hbm.at[rel, 0, pl.ds(row0, sz)], r2.at[pl.ds(0, sz)])

            @pl.loop(0, sz)
            def _(r):
                for c2 in range(2):
                    sl = pl.ds(c2 * 16, 16)
                    r1[r, sl] = r1[r, sl] * r2[r, sl]

            pltpu.sync_copy(r1.at[pl.ds(0, sz)], ycopy.at[pl.ds(row0, sz)])

        row_chunks(y0_chunk)

        pltpu.sync_copy(src_hbm.at[sid], sidx)
        pltpu.sync_copy(dst_hbm.at[sid], didx)

        def segsum_pass():
            for j in range(8):
                pltpu.async_copy(ycopy.at[sidx.at[j]], rows[j], gsem[j])

            @pl.loop(0, _CPC // 8 - 1)
            def _(g):
                for j in range(8):
                    c = g * 8 + j
                    pltpu.make_async_copy(ycopy.at[sidx.at[0]], rows[j], gsem[j]).wait()
                    pltpu.async_copy(rows[j], acc.at[didx.at[c]], ssem[j], add=True)
                for j in range(8):
                    c8 = g * 8 + 8 + j
                    pltpu.make_async_copy(rows[j], acc.at[didx.at[0]], ssem[j]).wait()
                    pltpu.async_copy(ycopy.at[sidx.at[c8]], rows[j], gsem[j])

            for j in range(8):
                c = _CPC - 8 + j
                pltpu.make_async_copy(ycopy.at[sidx.at[0]], rows[j], gsem[j]).wait()
                pltpu.async_copy(rows[j], acc.at[didx.at[c]], ssem[j], add=True)
            for j in range(8):
                pltpu.make_async_copy(rows[j], acc.at[didx.at[0]], ssem[j]).wait()

        plsc.subcore_barrier()
        segsum_pass()
        plsc.subcore_barrier()

        def mid_chunk(row0, sz):
            pltpu.sync_copy(acc.at[pl.ds(row0, sz)], r0.at[pl.ds(0, sz)])
            pltpu.sync_copy(ycopy.at[pl.ds(row0, sz)], r1.at[pl.ds(0, sz)])
            pltpu.sync_copy(dm_hbm.at[rel, 1, pl.ds(row0, sz)], r2.at[pl.ds(0, sz)])
            pltpu.sync_copy(dm_hbm.at[rel, 2, pl.ds(row0, sz)], r3.at[pl.ds(0, sz)])

            @pl.loop(0, sz)
            def _(r):
                for c2 in range(2):
                    sl = pl.ds(c2 * 16, 16)
                    y1 = r1[r, sl] - r0[r, sl] * r2[r, sl]
                    r1[r, sl] = y1
                    r0[r, sl] = y1 * r3[r, sl]

            pltpu.sync_copy(r1.at[pl.ds(0, sz)], ycopy.at[pl.ds(row0, sz)])
            pltpu.sync_copy(r0.at[pl.ds(0, sz)], f1_hbm.at[cid, pl.ds(row0, sz)])

        row_chunks(mid_chunk)
        zero_acc()
        plsc.subcore_barrier()
        segsum_pass()
        plsc.subcore_barrier()

        def out_chunk(row0, sz):
            pltpu.sync_copy(acc.at[pl.ds(row0, sz)], m2_hbm.at[cid, pl.ds(row0, sz)])

        row_chunks(out_chunk)

    return k(h2, dmeta, src3, dst3)


def _sc_deg(dst_both):
    @functools.partial(
        pl.kernel,
        out_type=jax.ShapeDtypeStruct((2, _N, 16), jnp.float32),
        mesh=_mesh(),
        scratch_types=[
            pltpu.VMEM((8, 128), jnp.int32),
            pltpu.VMEM((128, 16), jnp.float32),
            pltpu.VMEM_SHARED((_NPAD, 16), jnp.float32),
        ],
        compiler_params=_SC_PARAMS,
    )
    def k(dst_hbm, out_hbm, didx, rows, acc):
        cid = lax.axis_index("c")
        sid = lax.axis_index("s")
        zero = jnp.zeros((16,), jnp.float32)
        one = jnp.ones((16,), jnp.float32)

        @pl.loop(0, 128)
        def _(r):
            rows[r, pl.ds(0, 16)] = zero

        base = sid * 624
        for t in range(4):
            pltpu.sync_copy(rows, acc.at[pl.ds(base + t * 128, 128)])
        pltpu.sync_copy(rows.at[pl.ds(0, 112)], acc.at[pl.ds(base + 512, 112)])

        @pl.when(sid == 15)
        def _():
            pltpu.sync_copy(rows.at[pl.ds(0, 32)], acc.at[pl.ds(9984, 32)])

        @pl.loop(0, 128)
        def _(r):
            rows[r, pl.ds(0, 16)] = one

        plsc.subcore_barrier()

        @pl.loop(0, 20)
        def _(g):
            pltpu.sync_copy(dst_hbm.at[cid, sid, pl.ds(g * 8, 8)], didx)
            for j in range(8):
                pltpu.sync_copy(rows, acc.at[didx.at[j]], add=True)

        plsc.subcore_barrier()
        ob = sid * 624
        pltpu.sync_copy(acc.at[pl.ds(ob, 624)], out_hbm.at[cid, pl.ds(ob, 624)])

        @pl.when(sid == 15)
        def _():
            pltpu.sync_copy(acc.at[pl.ds(9984, 16)], out_hbm.at[cid, pl.ds(9984, 16)])

    return k(dst_both)


def _lrelu(x):
    return jnp.where(x >= 0, x, 0.01 * x)


def _dinv(deg_ref):
    return lax.rsqrt(jnp.maximum(deg_ref[:, 0:1], 1.0))


def _mlp(x, W1, b1, W2, b2):
    def body(x_ref, w1_ref, b1_ref, w2_ref, b2_ref, o_ref, h2_ref):
        h = jnp.dot(x_ref[...], w1_ref[...], preferred_element_type=jnp.float32, precision=lax.Precision.HIGHEST)
        h = _lrelu(h + b1_ref[...])
        h = jnp.dot(h, w2_ref[...], preferred_element_type=jnp.float32, precision=lax.Precision.HIGHEST)
        h = _lrelu(h + b2_ref[...])
        o_ref[...] = h
        h2_ref[0] = h[:, :32]
        h2_ref[1] = h[:, 32:]

    return pl.pallas_call(
        body,
        grid=(_N // _BLK,),
        in_specs=[
            pl.BlockSpec((_BLK, 128), lambda i: (i, 0)),
            pl.BlockSpec((128, _H), lambda i: (0, 0)),
            pl.BlockSpec((1, _H), lambda i: (0, 0)),
            pl.BlockSpec((_H, _H), lambda i: (0, 0)),
            pl.BlockSpec((1, _H), lambda i: (0, 0)),
        ],
        out_specs=[
            pl.BlockSpec((_BLK, _H), lambda i: (i, 0)),
            pl.BlockSpec((2, _BLK, 32), lambda i: (0, i, 0)),
        ],
        out_shape=[
            jax.ShapeDtypeStruct((_N, _H), jnp.float32),
            jax.ShapeDtypeStruct((2, _N, 32), jnp.float32),
        ],
    )(x, W1, b1.reshape(1, _H), W2, b2.reshape(1, _H))


def _dinvs(deg_both):
    def body(deg_ref, o_ref):
        for r in range(2):
            d = jnp.maximum(deg_ref[r, :, 0:1], 1.0)
            di = lax.rsqrt(d)
            o_ref[r, 0] = jnp.broadcast_to(di, (_N, 32))
            o_ref[r, 1] = jnp.broadcast_to(di * di, (_N, 32))
            o_ref[r, 2] = jnp.broadcast_to(d * di, (_N, 32))

    return pl.pallas_call(
        body,
        grid=(1,),
        in_specs=[pl.BlockSpec((2, _N, 16), lambda i: (0, 0, 0))],
        out_specs=pl.BlockSpec((2, 3, _N, 32), lambda i: (0, 0, 0, 0)),
        out_shape=jax.ShapeDtypeStruct((2, 3, _N, 32), jnp.float32),
    )(deg_both)


def _end(h, f1p, m2p, deg, W3s, b3):
    def body(h_ref, f1_ref, m2_ref, deg_ref, w3_ref, b3_ref, o_ref, h2_ref):
        di = _dinv(deg_ref)
        f1 = jnp.concatenate([f1_ref[0], f1_ref[1]], axis=-1)
        m2 = jnp.concatenate([m2_ref[0], m2_ref[1]], axis=-1)
        f2 = f1 - m2 * di
        o = jnp.dot(h_ref[...], w3_ref[0], preferred_element_type=jnp.float32, precision=lax.Precision.HIGHEST)
        o += jnp.dot(f1, w3_ref[1], preferred_element_type=jnp.float32, precision=lax.Precision.HIGHEST)
        o += jnp.dot(f2, w3_ref[2], preferred_element_type=jnp.float32, precision=lax.Precision.HIGHEST)
        o = o + b3_ref[...]
        o_ref[...] = o
        h2_ref[0] = o[:, :32]
        h2_ref[1] = o[:, 32:]

    return pl.pallas_call(
        body,
        grid=(_N // _BLK,),
        in_specs=[
            pl.BlockSpec((_BLK, _H), lambda i: (i, 0)),
            pl.BlockSpec((2, _BLK, 32), lambda i: (0, i, 0)),
            pl.BlockSpec((2, _BLK, 32), lambda i: (0, i, 0)),
            pl.BlockSpec((_BLK, 16), lambda i: (i, 0)),
            pl.BlockSpec((3, _H, _H), lambda i: (0, 0, 0)),
            pl.BlockSpec((1, _H), lambda i: (0, 0)),
        ],
        out_specs=[
            pl.BlockSpec((_BLK, _H), lambda i: (i, 0)),
            pl.BlockSpec((2, _BLK, 32), lambda i: (0, i, 0)),
        ],
        out_shape=[
            jax.ShapeDtypeStruct((_N, _H), jnp.float32),
            jax.ShapeDtypeStruct((2, _N, 32), jnp.float32),
        ],
    )(h, f1p, m2p, deg, W3s, b3)


def _final(ha, hb, W4p, b4p):
    def body(a_ref, b_ref, w_ref, bias_ref, o_ref):
        s = _lrelu(a_ref[...] + b_ref[...])
        o_ref[...] = (
            jnp.dot(s, w_ref[...], preferred_element_type=jnp.float32, precision=lax.Precision.HIGHEST)
            + bias_ref[...]
        )

    return pl.pallas_call(
        body,
        grid=(_N // _BLK,),
        in_specs=[
            pl.BlockSpec((_BLK, _H), lambda i: (i, 0)),
            pl.BlockSpec((_BLK, _H), lambda i: (i, 0)),
            pl.BlockSpec((_H, 128), lambda i: (0, 0)),
            pl.BlockSpec((1, 128), lambda i: (0, 0)),
        ],
        out_specs=pl.BlockSpec((_BLK, 128), lambda i: (i, 0)),
        out_shape=jax.ShapeDtypeStruct((_N, 128), jnp.float32),
    )(ha, hb, W4p, b4p)


def kernel(in_feat, edge_index_r0, edge_index_r1, W1, b1, W2, b2, W3, b3, W4, b4):
    pad = _EPAD - _E

    def prep_edges(ei):
        src = jnp.concatenate([ei[0], jnp.zeros((pad,), jnp.int32)])
        dst = jnp.concatenate([ei[1], jnp.full((pad,), _N, jnp.int32)])
        return src.reshape(16, _CPC, 128), dst.reshape(16, _CPC, 128)

    src0, dst0 = prep_edges(edge_index_r0)
    src1, dst1 = prep_edges(edge_index_r1)

    deg_both = _sc_deg(jnp.stack([dst0, dst1]))
    dmeta = _dinvs(deg_both)
    h, h2 = _mlp(in_feat, W1, b1, W2, b2)

    W3s = jnp.stack([
        3.0 * W3[:_H],
        -3.0 * W3[:_H] + 3.0 * W3[_H:2 * _H],
        0.75 * W3[:_H] - 1.5 * W3[_H:2 * _H] + 0.75 * W3[2 * _H:],
    ])
    b3r = b3.reshape(1, _H)

    h_all = []
    for r, (src3, dst3) in enumerate(((src0, dst0), (src1, dst1))):
        f1p, m2p = _sc_relation(h2, dmeta, src3, dst3, r)
        h, h2 = _end(h, f1p, m2p, deg_both[r], W3s, b3r)
        h_all.append(h)

    W4p = jnp.zeros((_H, 128), jnp.float32).at[:, :2].set(W4)
    b4p = jnp.zeros((1, 128), jnp.float32).at[0, :2].set(b4)
    out = _final(h_all[0], h_all[1], W4p, b4p)
    return out[:, :2]


# speedup vs baseline: 1.0456x; 1.0456x over previous
import functools

import jax
import jax.numpy as jnp
from jax import lax
from jax.experimental import pallas as pl
from jax.experimental.pallas import tpu as pltpu
from jax.experimental.pallas import tpu_sc as plsc

_N = 10000
_E = 320000
_H = 64
_NPAD = 10016
_EPAD = 327680
_NTILES = 32
_CPC = 160
_BLK = 1000


def _mesh():
    return plsc.VectorSubcoreMesh(core_axis_name="c", subcore_axis_name="s")


_SC_PARAMS = pltpu.CompilerParams(use_tc_tiling_on_sc=False)


def _sc_relation(y2, dinv2p, src3, dst3, rel):
    @functools.partial(
        pl.kernel,
        out_type=(
            jax.ShapeDtypeStruct((2, _N, 32), jnp.float32),
            jax.ShapeDtypeStruct((2, _N, 32), jnp.float32),
        ),
        mesh=_mesh(),
        scratch_types=[
            pltpu.VMEM((160, 128), jnp.int32),
            pltpu.VMEM((160, 128), jnp.int32),
            pltpu.VMEM((128, 32), jnp.float32),
            pltpu.VMEM((128, 32), jnp.float32),
            pltpu.VMEM((128, 32), jnp.float32),
            pltpu.VMEM((128, 32), jnp.float32),
            pltpu.VMEM((128, 32), jnp.float32),
            pltpu.VMEM((128, 32), jnp.float32),
            pltpu.VMEM((128, 32), jnp.float32),
            pltpu.VMEM((128, 32), jnp.float32),
            pltpu.VMEM_SHARED((_NPAD, 32), jnp.float32),
            pltpu.VMEM_SHARED((_N, 32), jnp.float32),
        ] + [pltpu.SemaphoreType.DMA] * 16,
        compiler_params=_SC_PARAMS,
    )
    def k(y_hbm, dm_hbm, src_hbm, dst_hbm, y1_hbm, m2_hbm, sidx, didx,
          r0, r1, r2, r3, r4, r5, r6, r7, acc, ycopy, *sems):
        cid = lax.axis_index("c")
        sid = lax.axis_index("s")
        rows = (r0, r1, r2, r3, r4, r5, r6, r7)
        gsem = sems[:8]
        ssem = sems[8:]
        zero = jnp.zeros((16,), jnp.float32)
        base = sid * 624

        def row_chunks(f):
            for t in range(4):
                f(base + t * 128, 128)
            f(base + 512, 112)

            @pl.when(sid == 15)
            def _():
                f(9984, 16)

        def zero_acc():
            @pl.loop(0, 128)
            def _(r):
                for c2 in range(2):
                    r0[r, pl.ds(c2 * 16, 16)] = zero

            def zc(row0, sz):
                pltpu.sync_copy(r0.at[pl.ds(0, sz)], acc.at[pl.ds(row0, sz)])

            row_chunks(zc)

        zero_acc()

        def stage_chunk(row0, sz):
            pltpu.sync_copy(y_hbm.at[cid, pl.ds(row0, sz)], ycopy.at[pl.ds(row0, sz)])

        row_chunks(stage_chunk)
        pltpu.sync_copy(src_hbm.at[sid], sidx)
        pltpu.sync_copy(dst_hbm.at[sid], didx)

        def segsum_pass():
            for j in range(8):
                pltpu.async_copy(ycopy.at[sidx.at[j]], rows[j], gsem[j])

            @pl.loop(0, _CPC // 8 - 1)
            def _(g):
                for j in range(8):
                    c = g * 8 + j
                    pltpu.make_async_copy(ycopy.at[sidx.at[0]], rows[j], gsem[j]).wait()
                    pltpu.async_copy(rows[j], acc.at[didx.at[c]], ssem[j], add=True)
                for j in range(8):
                    c8 = g * 8 + 8 + j
                    pltpu.make_async_copy(rows[j], acc.at[didx.at[0]], ssem[j]).wait()
                    pltpu.async_copy(ycopy.at[sidx.at[c8]], rows[j], gsem[j])

            for j in range(8):
                c = _CPC - 8 + j
                pltpu.make_async_copy(ycopy.at[sidx.at[0]], rows[j], gsem[j]).wait()
                pltpu.async_copy(rows[j], acc.at[didx.at[c]], ssem[j], add=True)
            for j in range(8):
                pltpu.make_async_copy(rows[j], acc.at[didx.at[0]], ssem[j]).wait()

        plsc.subcore_barrier()
        segsum_pass()
        plsc.subcore_barrier()

        def mid_chunk(row0, sz):
            pltpu.sync_copy(acc.at[pl.ds(row0, sz)], r0.at[pl.ds(0, sz)])
            pltpu.sync_copy(ycopy.at[pl.ds(row0, sz)], r1.at[pl.ds(0, sz)])
            pltpu.sync_copy(dm_hbm.at[rel, pl.ds(row0, sz)], r2.at[pl.ds(0, sz)])

            @pl.loop(0, sz)
            def _(r):
                for c2 in range(2):
                    sl = pl.ds(c2 * 16, 16)
                    r1[r, sl] = r1[r, sl] - r0[r, sl] * r2[r, sl]

            pltpu.sync_copy(r1.at[pl.ds(0, sz)], ycopy.at[pl.ds(row0, sz)])
            pltpu.sync_copy(r1.at[pl.ds(0, sz)], y1_hbm.at[cid, pl.ds(row0, sz)])

        row_chunks(mid_chunk)
        zero_acc()
        plsc.subcore_barrier()
        segsum_pass()
        plsc.subcore_barrier()

        def out_chunk(row0, sz):
            pltpu.sync_copy(acc.at[pl.ds(row0, sz)], m2_hbm.at[cid, pl.ds(row0, sz)])

        row_chunks(out_chunk)

    return k(y2, dinv2p, src3, dst3)


def _sc_deg(dst_both):
    @functools.partial(
        pl.kernel,
        out_type=jax.ShapeDtypeStruct((2, _N, 16), jnp.float32),
        mesh=_mesh(),
        scratch_types=[
            pltpu.VMEM((160, 128), jnp.int32),
            pltpu.VMEM((128, 16), jnp.float32),
            pltpu.VMEM((128, 16), jnp.float32),
            pltpu.VMEM_SHARED((_NPAD, 16), jnp.float32),
        ] + [pltpu.SemaphoreType.DMA] * 8,
        compiler_params=_SC_PARAMS,
    )
    def k(dst_hbm, out_hbm, didx, zrows, ones, acc, *sems):
        cid = lax.axis_index("c")
        sid = lax.axis_index("s")
        zero = jnp.zeros((16,), jnp.float32)
        one = jnp.ones((16,), jnp.float32)

        @pl.loop(0, 128)
        def _(r):
            zrows[r, pl.ds(0, 16)] = zero
            ones[r, pl.ds(0, 16)] = one

        base = sid * 624
        for t in range(4):
            pltpu.sync_copy(zrows, acc.at[pl.ds(base + t * 128, 128)])
        pltpu.sync_copy(zrows.at[pl.ds(0, 112)], acc.at[pl.ds(base + 512, 112)])

        @pl.when(sid == 15)
        def _():
            pltpu.sync_copy(zrows.at[pl.ds(0, 32)], acc.at[pl.ds(9984, 32)])

        pltpu.sync_copy(dst_hbm.at[cid, sid], didx)
        plsc.subcore_barrier()

        for j in range(8):
            pltpu.async_copy(ones, acc.at[didx.at[j]], sems[j], add=True)

        @pl.loop(0, _CPC // 8 - 1)
        def _(g):
            for j in range(8):
                c8 = g * 8 + 8 + j
                pltpu.make_async_copy(ones, acc.at[didx.at[0]], sems[j]).wait()
                pltpu.async_copy(ones, acc.at[didx.at[c8]], sems[j], add=True)

        for j in range(8):
            pltpu.make_async_copy(ones, acc.at[didx.at[0]], sems[j]).wait()

        plsc.subcore_barrier()
        ob = sid * 624
        pltpu.sync_copy(acc.at[pl.ds(ob, 624)], out_hbm.at[cid, pl.ds(ob, 624)])

        @pl.when(sid == 15)
        def _():
            pltpu.sync_copy(acc.at[pl.ds(9984, 16)], out_hbm.at[cid, pl.ds(9984, 16)])

    return k(dst_both)


def _lrelu(x):
    return jnp.where(x >= 0, x, 0.01 * x)


def _dinv(deg_ref):
    return lax.rsqrt(jnp.maximum(deg_ref[:, 0:1], 1.0))


def _mlp(x, W1, b1, W2, b2):
    def body(x_ref, w1_ref, b1_ref, w2_ref, b2_ref, o_ref):
        h = jnp.dot(x_ref[...], w1_ref[...], preferred_element_type=jnp.float32, precision=lax.Precision.HIGHEST)
        h = _lrelu(h + b1_ref[...])
        h = jnp.dot(h, w2_ref[...], preferred_element_type=jnp.float32, precision=lax.Precision.HIGHEST)
        o_ref[...] = _lrelu(h + b2_ref[...])

    return pl.pallas_call(
        body,
        grid=(_N // _BLK,),
        in_specs=[
            pl.BlockSpec((_BLK, 128), lambda i: (i, 0)),
            pl.BlockSpec((128, _H), lambda i: (0, 0)),
            pl.BlockSpec((1, _H), lambda i: (0, 0)),
            pl.BlockSpec((_H, _H), lambda i: (0, 0)),
            pl.BlockSpec((1, _H), lambda i: (0, 0)),
        ],
        out_specs=pl.BlockSpec((_BLK, _H), lambda i: (i, 0)),
        out_shape=jax.ShapeDtypeStruct((_N, _H), jnp.float32),
    )(x, W1, b1.reshape(1, _H), W2, b2.reshape(1, _H))


def _dinvs(deg_both):
    def body(deg_ref, o_ref):
        for r in range(2):
            d2 = 1.0 / jnp.maximum(deg_ref[r, :, 0:1], 1.0)
            o_ref[r] = jnp.broadcast_to(d2, (_N, 32))

    return pl.pallas_call(
        body,
        grid=(1,),
        in_specs=[pl.BlockSpec((2, _N, 16), lambda i: (0, 0, 0))],
        out_specs=pl.BlockSpec((2, _N, 32), lambda i: (0, 0, 0)),
        out_shape=jax.ShapeDtypeStruct((2, _N, 32), jnp.float32),
    )(deg_both)


def _prep(h, deg):
    def body(h_ref, deg_ref, y_ref):
        y = h_ref[...] * _dinv(deg_ref)
        y_ref[0] = y[:, :32]
        y_ref[1] = y[:, 32:]

    return pl.pallas_call(
        body,
        grid=(_N // _BLK,),
        in_specs=[
            pl.BlockSpec((_BLK, _H), lambda i: (i, 0)),
            pl.BlockSpec((_BLK, 16), lambda i: (i, 0)),
        ],
        out_specs=pl.BlockSpec((2, _BLK, 32), lambda i: (0, i, 0)),
        out_shape=jax.ShapeDtypeStruct((2, _N, 32), jnp.float32),
    )(h, deg)


def _end(h, y1p, m2p, deg, deg_next, W3s, b3):
    def body(h_ref, y1_ref, m2_ref, deg_ref, degn_ref, w3_ref, b3_ref, o_ref, y2_ref):
        d = jnp.maximum(deg_ref[:, 0:1], 1.0)
        di = lax.rsqrt(d)
        y1 = jnp.concatenate([y1_ref[0], y1_ref[1]], axis=-1)
        m2 = jnp.concatenate([m2_ref[0], m2_ref[1]], axis=-1)
        f1 = y1 * (d * di)
        f2 = f1 - m2 * di
        o = jnp.dot(h_ref[...], w3_ref[0], preferred_element_type=jnp.float32, precision=lax.Precision.HIGHEST)
        o += jnp.dot(f1, w3_ref[1], preferred_element_type=jnp.float32, precision=lax.Precision.HIGHEST)
        o += jnp.dot(f2, w3_ref[2], preferred_element_type=jnp.float32, precision=lax.Precision.HIGHEST)
        o = o + b3_ref[...]
        o_ref[...] = o
        y2n = o * _dinv(degn_ref)
        y2_ref[0] = y2n[:, :32]
        y2_ref[1] = y2n[:, 32:]

    return pl.pallas_call(
        body,
        grid=(_N // _BLK,),
        in_specs=[
            pl.BlockSpec((_BLK, _H), lambda i: (i, 0)),
            pl.BlockSpec((2, _BLK, 32), lambda i: (0, i, 0)),
            pl.BlockSpec((2, _BLK, 32), lambda i: (0, i, 0)),
            pl.BlockSpec((_BLK, 16), lambda i: (i, 0)),
            pl.BlockSpec((_BLK, 16), lambda i: (i, 0)),
            pl.BlockSpec((3, _H, _H), lambda i: (0, 0, 0)),
            pl.BlockSpec((1, _H), lambda i: (0, 0)),
        ],
        out_specs=[
            pl.BlockSpec((_BLK, _H), lambda i: (i, 0)),
            pl.BlockSpec((2, _BLK, 32), lambda i: (0, i, 0)),
        ],
        out_shape=[
            jax.ShapeDtypeStruct((_N, _H), jnp.float32),
            jax.ShapeDtypeStruct((2, _N, 32), jnp.float32),
        ],
    )(h, y1p, m2p, deg, deg_next, W3s, b3)


def _end_head(h, y1p, m2p, deg, W3s, b3, h0, W4p, b4p):
    def body(h_ref, y1_ref, m2_ref, deg_ref, w3_ref, b3_ref, h0_ref, w4_ref,
             b4_ref, o_ref):
        d = jnp.maximum(deg_ref[:, 0:1], 1.0)
        di = lax.rsqrt(d)
        y1 = jnp.concatenate([y1_ref[0], y1_ref[1]], axis=-1)
        m2 = jnp.concatenate([m2_ref[0], m2_ref[1]], axis=-1)
        f1 = y1 * (d * di)
        f2 = f1 - m2 * di
        o = jnp.dot(h_ref[...], w3_ref[0], preferred_element_type=jnp.float32, precision=lax.Precision.HIGHEST)
        o += jnp.dot(f1, w3_ref[1], preferred_element_type=jnp.float32, precision=lax.Precision.HIGHEST)
        o += jnp.dot(f2, w3_ref[2], preferred_element_type=jnp.float32, precision=lax.Precision.HIGHEST)
        s = _lrelu(h0_ref[...] + o + b3_ref[...])
        o_ref[...] = (
            jnp.dot(s, w4_ref[...], preferred_element_type=jnp.float32, precision=lax.Precision.HIGHEST)
            + b4_ref[...]
        )

    return pl.pallas_call(
        body,
        grid=(_N // _BLK,),
        in_specs=[
            pl.BlockSpec((_BLK, _H), lambda i: (i, 0)),
            pl.BlockSpec((2, _BLK, 32), lambda i: (0, i, 0)),
            pl.BlockSpec((2, _BLK, 32), lambda i: (0, i, 0)),
            pl.BlockSpec((_BLK, 16), lambda i: (i, 0)),
            pl.BlockSpec((3, _H, _H), lambda i: (0, 0, 0)),
            pl.BlockSpec((1, _H), lambda i: (0, 0)),
            pl.BlockSpec((_BLK, _H), lambda i: (i, 0)),
            pl.BlockSpec((_H, 128), lambda i: (0, 0)),
            pl.BlockSpec((1, 128), lambda i: (0, 0)),
        ],
        out_specs=pl.BlockSpec((_BLK, 128), lambda i: (i, 0)),
        out_shape=jax.ShapeDtypeStruct((_N, 128), jnp.float32),
    )(h, y1p, m2p, deg, W3s, b3, h0, W4p, b4p)


def kernel(in_feat, edge_index_r0, edge_index_r1, W1, b1, W2, b2, W3, b3, W4, b4):
    pad = _EPAD - _E

    def prep_edges(ei):
        src = jnp.concatenate([ei[0], jnp.zeros((pad,), jnp.int32)])
        dst = jnp.concatenate([ei[1], jnp.full((pad,), _N, jnp.int32)])
        return src.reshape(16, _CPC, 128), dst.reshape(16, _CPC, 128)

    src0, dst0 = prep_edges(edge_index_r0)
    src1, dst1 = prep_edges(edge_index_r1)

    deg_both = _sc_deg(jnp.stack([dst0, dst1]))
    dinv2p = _dinvs(deg_both)
    h = _mlp(in_feat, W1, b1, W2, b2)

    W3s = jnp.stack([
        3.0 * W3[:_H],
        -3.0 * W3[:_H] + 3.0 * W3[_H:2 * _H],
        0.75 * W3[:_H] - 1.5 * W3[_H:2 * _H] + 0.75 * W3[2 * _H:],
    ])
    b3r = b3.reshape(1, _H)
    W4p = jnp.zeros((_H, 128), jnp.float32).at[:, :2].set(W4)
    b4p = jnp.zeros((1, 128), jnp.float32).at[0, :2].set(b4)

    y2 = _prep(h, deg_both[0])
    y1p, m2p = _sc_relation(y2, dinv2p, src0, dst0, 0)
    h0, y2 = _end(h, y1p, m2p, deg_both[0], deg_both[1], W3s, b3r)
    y1p, m2p = _sc_relation(y2, dinv2p, src1, dst1, 1)
    out = _end_head(h0, y1p, m2p, deg_both[1], W3s, b3r, h0, W4p, b4p)
    return out[:, :2]


# speedup vs baseline: 1.1229x; 1.0739x over previous
import functools

import jax
import jax.numpy as jnp
from jax import lax
from jax.experimental import pallas as pl
from jax.experimental.pallas import tpu as pltpu
from jax.experimental.pallas import tpu_sc as plsc

_N = 10000
_E = 320000
_H = 64
_NPAD = 10016
_EPAD = 327680
_NTILES = 32
_CPC = 160
_BLK = 2000


def _mesh():
    return plsc.VectorSubcoreMesh(core_axis_name="c", subcore_axis_name="s")


_SC_PARAMS = pltpu.CompilerParams(use_tc_tiling_on_sc=False)


def _sc_relation(y2, dinv2p, src3, dst3, rel):
    @functools.partial(
        pl.kernel,
        out_type=(
            jax.ShapeDtypeStruct((2, _N, 32), jnp.float32),
            jax.ShapeDtypeStruct((2, _N, 32), jnp.float32),
        ),
        mesh=_mesh(),
        scratch_types=[
            pltpu.VMEM((160, 128), jnp.int32),
            pltpu.VMEM((160, 128), jnp.int32),
            pltpu.VMEM((128, 32), jnp.float32),
            pltpu.VMEM((128, 32), jnp.float32),
            pltpu.VMEM((128, 32), jnp.float32),
            pltpu.VMEM((128, 32), jnp.float32),
            pltpu.VMEM((128, 32), jnp.float32),
            pltpu.VMEM((128, 32), jnp.float32),
            pltpu.VMEM((128, 32), jnp.float32),
            pltpu.VMEM((128, 32), jnp.float32),
            pltpu.VMEM_SHARED((_NPAD, 32), jnp.float32),
            pltpu.VMEM_SHARED((_N, 32), jnp.float32),
        ] + [pltpu.SemaphoreType.DMA] * 16,
        compiler_params=_SC_PARAMS,
    )
    def k(y_hbm, dm_hbm, src_hbm, dst_hbm, y1_hbm, m2_hbm, sidx, didx,
          r0, r1, r2, r3, r4, r5, r6, r7, acc, ycopy, *sems):
        cid = lax.axis_index("c")
        sid = lax.axis_index("s")
        rows = (r0, r1, r2, r3, r4, r5, r6, r7)
        gsem = sems[:8]
        ssem = sems[8:]
        zero = jnp.zeros((16,), jnp.float32)
        base = sid * 624

        def row_chunks(f):
            for t in range(4):
                f(base + t * 128, 128)
            f(base + 512, 112)

            @pl.when(sid == 15)
            def _():
                f(9984, 16)

        def zero_acc():
            @pl.loop(0, 128)
            def _(r):
                for c2 in range(2):
                    r0[r, pl.ds(c2 * 16, 16)] = zero

            def zc(row0, sz):
                pltpu.sync_copy(r0.at[pl.ds(0, sz)], acc.at[pl.ds(row0, sz)])

            row_chunks(zc)

        zero_acc()

        def stage_chunk(row0, sz):
            pltpu.sync_copy(y_hbm.at[cid, pl.ds(row0, sz)], ycopy.at[pl.ds(row0, sz)])

        row_chunks(stage_chunk)
        pltpu.sync_copy(src_hbm.at[sid], sidx)
        pltpu.sync_copy(dst_hbm.at[sid], didx)

        def segsum_pass():
            for j in range(8):
                pltpu.async_copy(ycopy.at[sidx.at[j]], rows[j], gsem[j])

            @pl.loop(0, _CPC // 8 - 1)
            def _(g):
                for j in range(8):
                    c = g * 8 + j
                    pltpu.make_async_copy(ycopy.at[sidx.at[0]], rows[j], gsem[j]).wait()
                    pltpu.async_copy(rows[j], acc.at[didx.at[c]], ssem[j], add=True)
                for j in range(8):
                    c8 = g * 8 + 8 + j
                    pltpu.make_async_copy(rows[j], acc.at[didx.at[0]], ssem[j]).wait()
                    pltpu.async_copy(ycopy.at[sidx.at[c8]], rows[j], gsem[j])

            for j in range(8):
                c = _CPC - 8 + j
                pltpu.make_async_copy(ycopy.at[sidx.at[0]], rows[j], gsem[j]).wait()
                pltpu.async_copy(rows[j], acc.at[didx.at[c]], ssem[j], add=True)
            for j in range(8):
                pltpu.make_async_copy(rows[j], acc.at[didx.at[0]], ssem[j]).wait()

        plsc.subcore_barrier()
        segsum_pass()
        plsc.subcore_barrier()

        def mid_chunk(row0, sz):
            pltpu.sync_copy(acc.at[pl.ds(row0, sz)], r0.at[pl.ds(0, sz)])
            pltpu.sync_copy(ycopy.at[pl.ds(row0, sz)], r1.at[pl.ds(0, sz)])
            pltpu.sync_copy(dm_hbm.at[rel, pl.ds(row0, sz)], r2.at[pl.ds(0, sz)])

            @pl.loop(0, sz)
            def _(r):
                for c2 in range(2):
                    sl = pl.ds(c2 * 16, 16)
                    r1[r, sl] = r1[r, sl] - r0[r, sl] * r2[r, sl]

            pltpu.sync_copy(r1.at[pl.ds(0, sz)], ycopy.at[pl.ds(row0, sz)])
            pltpu.sync_copy(r1.at[pl.ds(0, sz)], y1_hbm.at[cid, pl.ds(row0, sz)])

        row_chunks(mid_chunk)
        zero_acc()
        plsc.subcore_barrier()
        segsum_pass()
        plsc.subcore_barrier()

        def out_chunk(row0, sz):
            pltpu.sync_copy(acc.at[pl.ds(row0, sz)], m2_hbm.at[cid, pl.ds(row0, sz)])

        row_chunks(out_chunk)

    return k(y2, dinv2p, src3, dst3)


def _sc_deg(dst_both):
    @functools.partial(
        pl.kernel,
        out_type=jax.ShapeDtypeStruct((2, _N, 16), jnp.float32),
        mesh=_mesh(),
        scratch_types=[
            pltpu.VMEM((160, 128), jnp.int32),
            pltpu.VMEM((128, 16), jnp.float32),
            pltpu.VMEM((128, 16), jnp.float32),
            pltpu.VMEM_SHARED((_NPAD, 16), jnp.float32),
        ] + [pltpu.SemaphoreType.DMA] * 8,
        compiler_params=_SC_PARAMS,
    )
    def k(dst_hbm, out_hbm, didx, zrows, ones, acc, *sems):
        cid = lax.axis_index("c")
        sid = lax.axis_index("s")
        zero = jnp.zeros((16,), jnp.float32)
        one = jnp.ones((16,), jnp.float32)

        @pl.loop(0, 128)
        def _(r):
            zrows[r, pl.ds(0, 16)] = zero
            ones[r, pl.ds(0, 16)] = one

        base = sid * 624
        for t in range(4):
            pltpu.sync_copy(zrows, acc.at[pl.ds(base + t * 128, 128)])
        pltpu.sync_copy(zrows.at[pl.ds(0, 112)], acc.at[pl.ds(base + 512, 112)])

        @pl.when(sid == 15)
        def _():
            pltpu.sync_copy(zrows.at[pl.ds(0, 32)], acc.at[pl.ds(9984, 32)])

        pltpu.sync_copy(dst_hbm.at[cid, sid], didx)
        plsc.subcore_barrier()

        for j in range(8):
            pltpu.async_copy(ones, acc.at[didx.at[j]], sems[j], add=True)

        @pl.loop(0, _CPC // 8 - 1)
        def _(g):
            for j in range(8):
                c8 = g * 8 + 8 + j
                pltpu.make_async_copy(ones, acc.at[didx.at[0]], sems[j]).wait()
                pltpu.async_copy(ones, acc.at[didx.at[c8]], sems[j], add=True)

        for j in range(8):
            pltpu.make_async_copy(ones, acc.at[didx.at[0]], sems[j]).wait()

        plsc.subcore_barrier()
        ob = sid * 624
        pltpu.sync_copy(acc.at[pl.ds(ob, 624)], out_hbm.at[cid, pl.ds(ob, 624)])

        @pl.when(sid == 15)
        def _():
            pltpu.sync_copy(acc.at[pl.ds(9984, 16)], out_hbm.at[cid, pl.ds(9984, 16)])

    return k(dst_both)


def _lrelu(x):
    return jnp.where(x >= 0, x, 0.01 * x)


def _dinv(deg_ref):
    return lax.rsqrt(jnp.maximum(deg_ref[:, 0:1], 1.0))


def _mlp(x, W1, b1, W2, b2):
    def body(x_ref, w1_ref, b1_ref, w2_ref, b2_ref, o_ref):
        h = jnp.dot(x_ref[...], w1_ref[...], preferred_element_type=jnp.float32, precision=lax.Precision.HIGHEST)
        h = _lrelu(h + b1_ref[...])
        h = jnp.dot(h, w2_ref[...], preferred_element_type=jnp.float32, precision=lax.Precision.HIGHEST)
        o_ref[...] = _lrelu(h + b2_ref[...])

    return pl.pallas_call(
        body,
        grid=(_N // _BLK,),
        in_specs=[
            pl.BlockSpec((_BLK, 128), lambda i: (i, 0)),
            pl.BlockSpec((128, _H), lambda i: (0, 0)),
            pl.BlockSpec((1, _H), lambda i: (0, 0)),
            pl.BlockSpec((_H, _H), lambda i: (0, 0)),
            pl.BlockSpec((1, _H), lambda i: (0, 0)),
        ],
        out_specs=pl.BlockSpec((_BLK, _H), lambda i: (i, 0)),
        out_shape=jax.ShapeDtypeStruct((_N, _H), jnp.float32),
    )(x, W1, b1.reshape(1, _H), W2, b2.reshape(1, _H))


def _dinvs(deg_both):
    def body(deg_ref, o_ref):
        for r in range(2):
            d2 = 1.0 / jnp.maximum(deg_ref[r, :, 0:1], 1.0)
            o_ref[r] = jnp.broadcast_to(d2, (_N, 32))

    return pl.pallas_call(
        body,
        grid=(1,),
        in_specs=[pl.BlockSpec((2, _N, 16), lambda i: (0, 0, 0))],
        out_specs=pl.BlockSpec((2, _N, 32), lambda i: (0, 0, 0)),
        out_shape=jax.ShapeDtypeStruct((2, _N, 32), jnp.float32),
    )(deg_both)


def _prep(h, deg):
    def body(h_ref, deg_ref, y_ref):
        y = h_ref[...] * _dinv(deg_ref)
        y_ref[0] = y[:, :32]
        y_ref[1] = y[:, 32:]

    return pl.pallas_call(
        body,
        grid=(_N // _BLK,),
        in_specs=[
            pl.BlockSpec((_BLK, _H), lambda i: (i, 0)),
            pl.BlockSpec((_BLK, 16), lambda i: (i, 0)),
        ],
        out_specs=pl.BlockSpec((2, _BLK, 32), lambda i: (0, i, 0)),
        out_shape=jax.ShapeDtypeStruct((2, _N, 32), jnp.float32),
    )(h, deg)


def _end(h, y1p, m2p, deg, deg_next, W3s, b3):
    def body(h_ref, y1_ref, m2_ref, deg_ref, degn_ref, w3_ref, b3_ref, o_ref, y2_ref):
        d = jnp.maximum(deg_ref[:, 0:1], 1.0)
        di = lax.rsqrt(d)
        y1 = jnp.concatenate([y1_ref[0], y1_ref[1]], axis=-1)
        m2 = jnp.concatenate([m2_ref[0], m2_ref[1]], axis=-1)
        f1 = y1 * (d * di)
        f2 = f1 - m2 * di
        o = jnp.dot(h_ref[...], w3_ref[0], preferred_element_type=jnp.float32, precision=lax.Precision.HIGHEST)
        o += jnp.dot(f1, w3_ref[1], preferred_element_type=jnp.float32, precision=lax.Precision.HIGHEST)
        o += jnp.dot(f2, w3_ref[2], preferred_element_type=jnp.float32, precision=lax.Precision.HIGHEST)
        o = o + b3_ref[...]
        o_ref[...] = o
        y2n = o * _dinv(degn_ref)
        y2_ref[0] = y2n[:, :32]
        y2_ref[1] = y2n[:, 32:]

    return pl.pallas_call(
        body,
        grid=(_N // _BLK,),
        in_specs=[
            pl.BlockSpec((_BLK, _H), lambda i: (i, 0)),
            pl.BlockSpec((2, _BLK, 32), lambda i: (0, i, 0)),
            pl.BlockSpec((2, _BLK, 32), lambda i: (0, i, 0)),
            pl.BlockSpec((_BLK, 16), lambda i: (i, 0)),
            pl.BlockSpec((_BLK, 16), lambda i: (i, 0)),
            pl.BlockSpec((3, _H, _H), lambda i: (0, 0, 0)),
            pl.BlockSpec((1, _H), lambda i: (0, 0)),
        ],
        out_specs=[
            pl.BlockSpec((_BLK, _H), lambda i: (i, 0)),
            pl.BlockSpec((2, _BLK, 32), lambda i: (0, i, 0)),
        ],
        out_shape=[
            jax.ShapeDtypeStruct((_N, _H), jnp.float32),
            jax.ShapeDtypeStruct((2, _N, 32), jnp.float32),
        ],
    )(h, y1p, m2p, deg, deg_next, W3s, b3)


def _end_head(h, y1p, m2p, deg, W3s, b3, h0, W4p, b4p):
    def body(h_ref, y1_ref, m2_ref, deg_ref, w3_ref, b3_ref, h0_ref, w4_ref,
             b4_ref, o_ref):
        d = jnp.maximum(deg_ref[:, 0:1], 1.0)
        di = lax.rsqrt(d)
        y1 = jnp.concatenate([y1_ref[0], y1_ref[1]], axis=-1)
        m2 = jnp.concatenate([m2_ref[0], m2_ref[1]], axis=-1)
        f1 = y1 * (d * di)
        f2 = f1 - m2 * di
        o = jnp.dot(h_ref[...], w3_ref[0], preferred_element_type=jnp.float32, precision=lax.Precision.HIGHEST)
        o += jnp.dot(f1, w3_ref[1], preferred_element_type=jnp.float32, precision=lax.Precision.HIGHEST)
        o += jnp.dot(f2, w3_ref[2], preferred_element_type=jnp.float32, precision=lax.Precision.HIGHEST)
        s = _lrelu(h0_ref[...] + o + b3_ref[...])
        o_ref[...] = (
            jnp.dot(s, w4_ref[...], preferred_element_type=jnp.float32, precision=lax.Precision.HIGHEST)
            + b4_ref[...]
        )

    return pl.pallas_call(
        body,
        grid=(_N // _BLK,),
        in_specs=[
            pl.BlockSpec((_BLK, _H), lambda i: (i, 0)),
            pl.BlockSpec((2, _BLK, 32), lambda i: (0, i, 0)),
            pl.BlockSpec((2, _BLK, 32), lambda i: (0, i, 0)),
            pl.BlockSpec((_BLK, 16), lambda i: (i, 0)),
            pl.BlockSpec((3, _H, _H), lambda i: (0, 0, 0)),
            pl.BlockSpec((1, _H), lambda i: (0, 0)),
            pl.BlockSpec((_BLK, _H), lambda i: (i, 0)),
            pl.BlockSpec((_H, 128), lambda i: (0, 0)),
            pl.BlockSpec((1, 128), lambda i: (0, 0)),
        ],
        out_specs=pl.BlockSpec((_BLK, 128), lambda i: (i, 0)),
        out_shape=jax.ShapeDtypeStruct((_N, 128), jnp.float32),
    )(h, y1p, m2p, deg, W3s, b3, h0, W4p, b4p)


def kernel(in_feat, edge_index_r0, edge_index_r1, W1, b1, W2, b2, W3, b3, W4, b4):
    pad = _EPAD - _E

    def prep_edges(ei):
        src = jnp.concatenate([ei[0], jnp.zeros((pad,), jnp.int32)])
        dst = jnp.concatenate([ei[1], jnp.full((pad,), _N, jnp.int32)])
        return src.reshape(16, _CPC, 128), dst.reshape(16, _CPC, 128)

    src0, dst0 = prep_edges(edge_index_r0)
    src1, dst1 = prep_edges(edge_index_r1)

    deg_both = _sc_deg(jnp.stack([dst0, dst1]))
    dinv2p = _dinvs(deg_both)
    h = _mlp(in_feat, W1, b1, W2, b2)

    W3s = jnp.stack([
        3.0 * W3[:_H],
        -3.0 * W3[:_H] + 3.0 * W3[_H:2 * _H],
        0.75 * W3[:_H] - 1.5 * W3[_H:2 * _H] + 0.75 * W3[2 * _H:],
    ])
    b3r = b3.reshape(1, _H)
    W4p = jnp.zeros((_H, 128), jnp.float32).at[:, :2].set(W4)
    b4p = jnp.zeros((1, 128), jnp.float32).at[0, :2].set(b4)

    y2 = _prep(h, deg_both[0])
    y1p, m2p = _sc_relation(y2, dinv2p, src0, dst0, 0)
    h0, y2 = _end(h, y1p, m2p, deg_both[0], deg_both[1], W3s, b3r)
    y1p, m2p = _sc_relation(y2, dinv2p, src1, dst1, 1)
    out = _end_head(h0, y1p, m2p, deg_both[1], W3s, b3r, h0, W4p, b4p)
    return out[:, :2]
